# baseline (speedup 1.0000x reference)
import functools

import jax
import jax.numpy as jnp
from jax import lax
from jax.experimental import pallas as pl
from jax.experimental.pallas import tpu as pltpu
from jax.experimental.pallas import tpu_sc as plsc

WORLD = 8
NC, NS = 2, 16
NW = NC * NS
SPAD = 56
MS = 4
NBUF = 4
ZROWS = 256
SPLIT_S = 24


def _make_sc_call(R, S, V, D):
    local_vocab = V // WORLD
    bpw = R // NW
    nchunk_all = SPAD // MS
    nfull = S // MS
    srem = S - nfull * MS
    half = srem * bpw
    rows_per_s = (R // 8) * 64
    assert bpw == ZROWS // 8 and srem > 0 and nfull + 1 <= nchunk_all

    mesh = plsc.VectorSubcoreMesh(core_axis_name="c", subcore_axis_name="s",
                                  num_cores=NC, num_subcores=NS)

    @functools.partial(
        pl.kernel,
        out_type=jax.ShapeDtypeStruct((R * S * WORLD, D), jnp.float32),
        mesh=mesh,
        scratch_types=[
            pltpu.VMEM((nchunk_all, MS * bpw), jnp.int32),
            pltpu.VMEM((nchunk_all, MS * bpw), jnp.int32),
            pltpu.VMEM((1, half), jnp.int32),
            pltpu.VMEM((1, half), jnp.int32),
            pltpu.VMEM((ZROWS, D), jnp.float32),
            [pltpu.VMEM((MS * bpw, D), jnp.float32) for _ in range(NBUF)],
            [pltpu.SemaphoreType.DMA for _ in range(2)],
            [pltpu.SemaphoreType.DMA for _ in range(NBUF)],
            [pltpu.SemaphoreType.DMA for _ in range(NBUF)],
        ],
    )
    def sc_call(ids_hbm, table_hbm, zeros_hbm, out_hbm,
                ids_v, dst_v, ids_t, dst_t, zbuf, rows, sem_z, sem_g, sem_w):
        wid = lax.axis_index("s") * NC + lax.axis_index("c")
        b0 = wid * bpw

        pltpu.sync_copy(zeros_hbm, zbuf)
        zbase = (b0 // 8) * 64

        def zissue0(s, carry):
            pltpu.async_copy(
                zbuf, out_hbm.at[pl.ds(s * rows_per_s + zbase, ZROWS)],
                sem_z[0])
            return carry

        def zissue1(s, carry):
            pltpu.async_copy(
                zbuf, out_hbm.at[pl.ds(s * rows_per_s + zbase, ZROWS)],
                sem_z[1])
            return carry

        lax.fori_loop(0, SPLIT_S, zissue0, 0)
        lax.fori_loop(SPLIT_S, S, zissue1, 0)

        pltpu.sync_copy(ids_hbm.at[wid], ids_v)

        lane = lax.broadcasted_iota(jnp.int32, (16,), 0)
        base16 = (jnp.int32(b0 // 8) + lax.div(lane, jnp.int32(8))) * 64 \
            + lax.rem(lane, jnp.int32(8))

        def dbody(c, carry):
            for g in range(MS * bpw // 16):
                v = ids_v[c, pl.ds(g * 16, 16)]
                seg = lax.div(v, jnp.int32(local_vocab))
                s = c * MS + g // 2
                base = base16 + jnp.int32((g % 2) * 2 * 64)
                dst_v[c, pl.ds(g * 16, 16)] = (
                    s * rows_per_s + base + seg * WORLD)
            return carry

        lax.fori_loop(0, nfull, dbody, 0)

        for g in range(half // 16):
            v = ids_v[nfull, pl.ds(g * 16, 16)]
            ids_t[0, pl.ds(g * 16, 16)] = v
            seg = lax.div(v, jnp.int32(local_vocab))
            s = nfull * MS + g // 2
            base = base16 + jnp.int32((g % 2) * 2 * 64)
            dst_t[0, pl.ds(g * 16, 16)] = s * rows_per_s + base + seg * WORLD

        def gath(c):
            pltpu.async_copy(table_hbm.at[ids_v.at[c]], rows[c % NBUF],
                             sem_g[c % NBUF])

        def wait_gath(c):
            pltpu.make_async_copy(table_hbm.at[ids_v.at[c]], rows[c % NBUF],
                                  sem_g[c % NBUF]).wait()

        def put(c):
            pltpu.async_copy(rows[c % NBUF], out_hbm.at[dst_v.at[c]],
                             sem_w[c % NBUF])

        def drain(c):
            pltpu.make_async_copy(rows[c % NBUF], out_hbm.at[dst_v.at[c]],
                                  sem_w[c % NBUF]).wait()

        for c in range(min(NBUF - 1, nfull)):
            gath(c)

        def zdrain(lo, hi, z):
            def body(s, carry):
                pltpu.make_async_copy(
                    zbuf, out_hbm.at[pl.ds(s * rows_per_s + zbase, ZROWS)],
                    sem_z[z]).wait()
                return carry
            lax.fori_loop(lo, hi, body, 0)

        for c in range(nfull):
            if c == 0:
                zdrain(0, SPLIT_S, 0)
            if c == SPLIT_S // MS:
                zdrain(SPLIT_S, S, 1)
            wait_gath(c)
            put(c)
            if c > 0:
                drain(c - 1)
            if c + NBUF - 1 < nfull:
                gath(c + NBUF - 1)
        drain(nfull - 1)

        tail = rows[0].at[pl.ds(0, half)]
        pltpu.async_copy(table_hbm.at[ids_t.at[0]], tail, sem_g[0]).wait()
        pltpu.async_copy(tail, out_hbm.at[dst_t.at[0]], sem_w[0]).wait()

    return sc_call


def kernel(input_ids, weight):
    R, S = input_ids.shape
    V, D = weight.shape
    ids = input_ids.astype(jnp.int32)
    ids_pad = jnp.pad(ids, ((0, 0), (0, SPAD - S)))
    ids_t3 = jnp.transpose(ids_pad.reshape(NW, R // NW, SPAD), (0, 2, 1))
    ids_t3 = ids_t3.reshape(NW, SPAD // MS, MS * (R // NW))
    zeros = jnp.zeros((ZROWS, D), jnp.float32)
    flat = _make_sc_call(R, S, V, D)(ids_t3, weight, zeros)
    t = flat.reshape(S, R // 8, WORLD, 8, D)
    return t.transpose(1, 3, 0, 2, 4).reshape(R, S, WORLD * D)


# speedup vs baseline: 1.0198x; 1.0198x over previous
import functools

import jax
import jax.numpy as jnp
from jax import lax
from jax.experimental import pallas as pl
from jax.experimental.pallas import tpu as pltpu
from jax.experimental.pallas import tpu_sc as plsc

WORLD = 8
NC, NS = 2, 16
NW = NC * NS
SPAD = 56
MS = 4
NBUF = 4
ZROWS = 256


def _make_sc_call(R, S, V, D):
    local_vocab = V // WORLD
    bpw = R // NW
    nchunk_all = SPAD // MS
    nfull = S // MS
    srem = S - nfull * MS
    half = srem * bpw
    rows_per_s = (R // 8) * 64
    assert bpw == ZROWS // 8 and srem > 0 and nfull + 1 <= nchunk_all

    mesh = plsc.VectorSubcoreMesh(core_axis_name="c", subcore_axis_name="s",
                                  num_cores=NC, num_subcores=NS)

    @functools.partial(
        pl.kernel,
        out_type=jax.ShapeDtypeStruct((R * S * WORLD, D), jnp.float32),
        mesh=mesh,
        scratch_types=[
            pltpu.VMEM((nchunk_all, MS * bpw), jnp.int32),
            pltpu.VMEM((nchunk_all, MS * bpw), jnp.int32),
            pltpu.VMEM((1, half), jnp.int32),
            pltpu.VMEM((1, half), jnp.int32),
            pltpu.VMEM((ZROWS, D), jnp.float32),
            [pltpu.VMEM((MS * bpw, D), jnp.float32) for _ in range(NBUF)],
            pltpu.SemaphoreType.DMA,
            [pltpu.SemaphoreType.DMA for _ in range(NBUF)],
            [pltpu.SemaphoreType.DMA for _ in range(NBUF)],
        ],
    )
    def sc_call(ids_hbm, table_hbm, zeros_hbm, out_hbm,
                ids_v, dst_v, ids_t, dst_t, zbuf, rows, sem_z, sem_g, sem_w):
        wid = lax.axis_index("s") * NC + lax.axis_index("c")
        b0 = wid * bpw

        pltpu.sync_copy(zeros_hbm, zbuf)
        zbase = (b0 // 8) * 64

        def zissue(s, carry):
            pltpu.async_copy(
                zbuf, out_hbm.at[pl.ds(s * rows_per_s + zbase, ZROWS)], sem_z)
            return carry

        lax.fori_loop(0, S, zissue, 0)

        pltpu.sync_copy(ids_hbm.at[wid], ids_v)

        lane = lax.broadcasted_iota(jnp.int32, (16,), 0)
        base16 = (jnp.int32(b0 // 8) + lax.div(lane, jnp.int32(8))) * 64 \
            + lax.rem(lane, jnp.int32(8))

        def dbody(c, carry):
            for g in range(MS * bpw // 16):
                v = ids_v[c, pl.ds(g * 16, 16)]
                seg = lax.div(v, jnp.int32(local_vocab))
                s = c * MS + g // 2
                base = base16 + jnp.int32((g % 2) * 2 * 64)
                dst_v[c, pl.ds(g * 16, 16)] = (
                    s * rows_per_s + base + seg * WORLD)
            return carry

        lax.fori_loop(0, nfull, dbody, 0)

        for g in range(half // 16):
            v = ids_v[nfull, pl.ds(g * 16, 16)]
            ids_t[0, pl.ds(g * 16, 16)] = v
            seg = lax.div(v, jnp.int32(local_vocab))
            s = nfull * MS + g // 2
            base = base16 + jnp.int32((g % 2) * 2 * 64)
            dst_t[0, pl.ds(g * 16, 16)] = s * rows_per_s + base + seg * WORLD

        def gath(c):
            pltpu.async_copy(table_hbm.at[ids_v.at[c]], rows[c % NBUF],
                             sem_g[c % NBUF])

        def wait_gath(c):
            pltpu.make_async_copy(table_hbm.at[ids_v.at[c]], rows[c % NBUF],
                                  sem_g[c % NBUF]).wait()

        def put(c):
            pltpu.async_copy(rows[c % NBUF], out_hbm.at[dst_v.at[c]],
                             sem_w[c % NBUF])

        def drain(c):
            pltpu.make_async_copy(rows[c % NBUF], out_hbm.at[dst_v.at[c]],
                                  sem_w[c % NBUF]).wait()

        for c in range(min(NBUF - 1, nfull)):
            gath(c)

        def zdrain(s, carry):
            pltpu.make_async_copy(
                zbuf, out_hbm.at[pl.ds(s * rows_per_s + zbase, ZROWS)],
                sem_z).wait()
            return carry

        lax.fori_loop(0, S, zdrain, 0)

        for c in range(nfull):
            wait_gath(c)
            put(c)
            if c > 0:
                drain(c - 1)
            if c + NBUF - 1 < nfull:
                gath(c + NBUF - 1)
        drain(nfull - 1)

        tail = rows[0].at[pl.ds(0, half)]
        pltpu.async_copy(table_hbm.at[ids_t.at[0]], tail, sem_g[0]).wait()
        pltpu.async_copy(tail, out_hbm.at[dst_t.at[0]], sem_w[0]).wait()

    return sc_call


def kernel(input_ids, weight):
    R, S = input_ids.shape
    V, D = weight.shape
    ids = input_ids.astype(jnp.int32)
    ids_pad = jnp.pad(ids, ((0, 0), (0, SPAD - S)))
    ids_t3 = jnp.transpose(ids_pad.reshape(NW, R // NW, SPAD), (0, 2, 1))
    ids_t3 = ids_t3.reshape(NW, SPAD // MS, MS * (R // NW))
    zeros = jnp.zeros((ZROWS, D), jnp.float32)
    flat = _make_sc_call(R, S, V, D)(ids_t3, weight, zeros)
    t = flat.reshape(S, R // 8, WORLD, 8, D)
    return t.transpose(1, 3, 0, 2, 4).reshape(R, S, WORLD * D)
